# baseline (speedup 1.0000x reference)
import functools

import jax
import jax.numpy as jnp
from jax import lax
from jax.experimental import pallas as pl
from jax.experimental.pallas import tpu as pltpu
from jax.experimental.pallas import tpu_sc as plsc

N = 10000
DIM = 256
E = 160000
HID = DIM * 4
NEG = -1.0e9

ROWS_BLK = 1000

NW = 32
PR = 320
NPAD = NW * PR
CAP = 6848
SPILL = PR
GK = 32


def _gathermax_body(xn_hbm, ccol_hbm, crow_hbm, cnt_hbm, out_hbm,
                    acc, ccol, crow, gbufa, gbufb, cbuf16, sema, semb):
    wid = lax.axis_index("s") * 2 + lax.axis_index("c")
    lo = wid * PR
    neg16 = jnp.full((16,), NEG, jnp.float32)

    pltpu.sync_copy(ccol_hbm.at[pl.ds(wid * CAP, CAP)], ccol)
    pltpu.sync_copy(crow_hbm.at[pl.ds(wid * CAP, CAP)], crow)
    pltpu.sync_copy(cnt_hbm.at[pl.ds(wid * 16, 16)], cbuf16)
    pos = cbuf16[pl.ds(0, 16)][0]

    nch = lax.max((pos + GK - 1) // GK, 1)
    last = nch - 1
    npairs = (nch + 1) // 2

    def issue(c, buf, sem):
        pltpu.async_copy(xn_hbm.at[ccol.at[pl.ds(c * GK, GK)]], buf, sem)

    def drain(buf, sem):
        pltpu.make_async_copy(xn_hbm.at[pl.ds(0, GK)], buf, sem).wait()

    def accum(g, buf):
        def edge(e, _):
            rl = crow[pl.ds(g * GK + e, 16)][0]
            for j in range(DIM // 16):
                sl = pl.ds(j * 16, 16)
                acc[rl, sl] = jnp.maximum(acc[rl, sl], buf[e, sl])
            return 0
        lax.fori_loop(0, GK, edge, 0)

    issue(0, gbufa, sema)

    def init_r(r, _):
        for j in range(DIM // 16):
            acc[r, pl.ds(j * 16, 16)] = neg16
        return 0
    lax.fori_loop(0, PR + 1, init_r, 0)

    def pair(i, _):
        g0 = 2 * i
        g1 = g0 + 1
        issue(lax.min(g1, last), gbufb, semb)
        drain(gbufa, sema)
        accum(g0, gbufa)
        issue(lax.min(g0 + 2, last), gbufa, sema)
        drain(gbufb, semb)
        accum(g1, gbufb)
        return 0
    lax.fori_loop(0, npairs, pair, 0)
    drain(gbufa, sema)

    pltpu.sync_copy(acc.at[pl.ds(0, PR)], out_hbm.at[pl.ds(lo, PR)])


_gathermax = functools.partial(
    pl.kernel,
    out_type=jax.ShapeDtypeStruct((NPAD, DIM), jnp.float32),
    mesh=plsc.VectorSubcoreMesh(core_axis_name="c", subcore_axis_name="s"),
    scratch_types=[
        pltpu.VMEM((PR + 1, DIM), jnp.float32),
        pltpu.VMEM((CAP,), jnp.int32),
        pltpu.VMEM((CAP,), jnp.int32),
        pltpu.VMEM((GK, DIM), jnp.float32),
        pltpu.VMEM((GK, DIM), jnp.float32),
        pltpu.VMEM((16,), jnp.int32),
        pltpu.SemaphoreType.DMA,
        pltpu.SemaphoreType.DMA,
    ],
)(_gathermax_body)


def _aggregate(xn, row, col):
    bucket = row // PR
    word = (bucket << 23) | ((row - bucket * PR) << 14) | col
    sword = jnp.sort(word)
    sbucket = sword >> 23
    counts = jnp.bincount(bucket, length=NW)
    starts = jnp.concatenate([jnp.zeros((1,), jnp.int32),
                              jnp.cumsum(counts)[:-1].astype(jnp.int32)])
    pos = jnp.arange(E, dtype=jnp.int32) - starts[sbucket] + sbucket * CAP
    ccol = jnp.zeros((NW * CAP,), jnp.int32).at[pos].set(
        sword & 0x3FFF, mode="drop")
    crow = jnp.full((NW * CAP,), SPILL, jnp.int32).at[pos].set(
        (sword >> 14) & 0x1FF, mode="drop")
    cnt16 = jnp.repeat(jnp.minimum(counts, CAP).astype(jnp.int32), 16)
    return _gathermax(xn, ccol, crow, cnt16)


def _gelu_exact(x):
    return 0.5 * x * (1.0 + lax.erf(x * 0.7071067811865476))


def _ln(x, g, b, eps=1e-5):
    mu = jnp.mean(x, axis=-1, keepdims=True)
    var = jnp.mean((x - mu) ** 2, axis=-1, keepdims=True)
    return (x - mu) * lax.rsqrt(var + eps) * g + b


def _ln1_body(x_ref, g_ref, b_ref, o_ref):
    o_ref[...] = _ln(x_ref[...], g_ref[...], b_ref[...])


def _ln1(x2d, g1, be1):
    grid = (N // ROWS_BLK,)
    return pl.pallas_call(
        _ln1_body,
        grid=grid,
        in_specs=[
            pl.BlockSpec((ROWS_BLK, DIM), lambda i: (i, 0)),
            pl.BlockSpec((DIM,), lambda i: (0,)),
            pl.BlockSpec((DIM,), lambda i: (0,)),
        ],
        out_specs=pl.BlockSpec((ROWS_BLK, DIM), lambda i: (i, 0)),
        out_shape=jax.ShapeDtypeStruct((N, DIM), jnp.float32),
    )(x2d, g1, be1)


def _tail_body(aggr_ref, xn_ref, x_ref, W1_ref, bW1_ref, g2_ref, be2_ref,
               Wf1_ref, bf1_ref, Wf2_ref, bf2_ref, o_ref):
    aggr = aggr_ref[...]
    a = jnp.where(aggr == NEG, 0.0, aggr) - xn_ref[...]
    h = _gelu_exact(
        jnp.dot(a, W1_ref[...], preferred_element_type=jnp.float32) + bW1_ref[...])
    x1 = h + x_ref[...]
    xn2 = _ln(x1, g2_ref[...], be2_ref[...])
    hh = _gelu_exact(
        jnp.dot(xn2, Wf1_ref[...], preferred_element_type=jnp.float32) + bf1_ref[...])
    ff = jnp.dot(hh, Wf2_ref[...], preferred_element_type=jnp.float32) + bf2_ref[...]
    o_ref[...] = ff + x1


def _tail(aggr, xn, x2d, W1, bW1, g2, be2, Wf1, bf1, Wf2, bf2):
    grid = (N // ROWS_BLK,)
    row_spec = pl.BlockSpec((ROWS_BLK, DIM), lambda i: (i, 0))
    full = lambda shape: pl.BlockSpec(shape, lambda i: (0,) * len(shape))
    return pl.pallas_call(
        _tail_body,
        grid=grid,
        in_specs=[
            row_spec, row_spec, row_spec,
            full((DIM, DIM)), full((DIM,)), full((DIM,)), full((DIM,)),
            full((DIM, HID)), full((HID,)), full((HID, DIM)), full((DIM,)),
        ],
        out_specs=row_spec,
        out_shape=jax.ShapeDtypeStruct((N, DIM), jnp.float32),
    )(aggr, xn, x2d, W1, bW1, g2, be2, Wf1, bf1, Wf2, bf2)


def kernel(x, edge_index, g1, be1, W1, bW1, g2, be2, Wf1, bf1, Wf2, bf2):
    x2d = x.reshape(N, DIM)
    xn = _ln1(x2d, g1, be1)
    aggr = _aggregate(xn, edge_index[0], edge_index[1])[:N]
    out = _tail(aggr, xn, x2d, W1, bW1, g2, be2, Wf1, bf1, Wf2, bf2)
    return out.reshape(1, N, DIM)


# speedup vs baseline: 1.0906x; 1.0906x over previous
import functools

import jax
import jax.numpy as jnp
from jax import lax
from jax.experimental import pallas as pl
from jax.experimental.pallas import tpu as pltpu
from jax.experimental.pallas import tpu_sc as plsc

N = 10000
DIM = 256
E = 160000
HID = DIM * 4
NEG = -1.0e9

ROWS_BLK = 1000

NW = 32
PR = 320
NPAD = NW * PR
CAP = 6848
SPILL = PR
GK = 32


def _gathermax_body(xn_hbm, ccol_hbm, crow_hbm, cnt_hbm, out_hbm,
                    acc, ccol, crow, gbufa, gbufb, cbuf16, sema, semb):
    wid = lax.axis_index("s") * 2 + lax.axis_index("c")
    lo = wid * PR
    neg16 = jnp.full((16,), NEG, jnp.float32)

    pltpu.sync_copy(ccol_hbm.at[pl.ds(wid * CAP, CAP)], ccol)
    pltpu.sync_copy(crow_hbm.at[pl.ds(wid * CAP, CAP)], crow)
    pltpu.sync_copy(cnt_hbm.at[pl.ds(wid * 16, 16)], cbuf16)
    pos = cbuf16[pl.ds(0, 16)][0]

    nch = lax.max((pos + GK - 1) // GK, 1)
    last = nch - 1
    npairs = (nch + 1) // 2

    def issue(c, buf, sem):
        pltpu.async_copy(xn_hbm.at[ccol.at[pl.ds(c * GK, GK)]], buf, sem)

    def drain(buf, sem):
        pltpu.make_async_copy(xn_hbm.at[pl.ds(0, GK)], buf, sem).wait()

    def accum(g, buf):
        def edge(e, _):
            rl = crow[pl.ds(g * GK + e, 16)][0]
            for j in range(DIM // 16):
                sl = pl.ds(j * 16, 16)
                acc[rl, sl] = jnp.maximum(acc[rl, sl], buf[e, sl])
            return 0
        lax.fori_loop(0, GK, edge, 0)

    issue(0, gbufa, sema)

    def init_r(r, _):
        for j in range(DIM // 16):
            acc[r, pl.ds(j * 16, 16)] = neg16
        return 0
    lax.fori_loop(0, PR + 1, init_r, 0)

    def pair(i, _):
        g0 = 2 * i
        g1 = g0 + 1
        issue(lax.min(g1, last), gbufb, semb)
        drain(gbufa, sema)
        accum(g0, gbufa)
        issue(lax.min(g0 + 2, last), gbufa, sema)
        drain(gbufb, semb)
        accum(g1, gbufb)
        return 0
    lax.fori_loop(0, npairs, pair, 0)
    drain(gbufa, sema)

    pltpu.sync_copy(acc.at[pl.ds(0, PR)], out_hbm.at[pl.ds(lo, PR)])


_gathermax = functools.partial(
    pl.kernel,
    out_type=jax.ShapeDtypeStruct((NPAD, DIM), jnp.float32),
    mesh=plsc.VectorSubcoreMesh(core_axis_name="c", subcore_axis_name="s"),
    scratch_types=[
        pltpu.VMEM((PR + 1, DIM), jnp.float32),
        pltpu.VMEM((CAP,), jnp.int32),
        pltpu.VMEM((CAP,), jnp.int32),
        pltpu.VMEM((GK, DIM), jnp.float32),
        pltpu.VMEM((GK, DIM), jnp.float32),
        pltpu.VMEM((16,), jnp.int32),
        pltpu.SemaphoreType.DMA,
        pltpu.SemaphoreType.DMA,
    ],
)(_gathermax_body)


def _aggregate(xn, row, col):
    B2 = 128
    B1 = E // B2
    bucket = row // PR
    oh = (bucket.reshape(B1, B2)[:, :, None]
          == jnp.arange(NW)[None, None, :]).astype(jnp.float32)
    lt = (jnp.arange(B2)[:, None] > jnp.arange(B2)[None, :]).astype(jnp.float32)
    within = jnp.einsum("ij,bjk->bik", lt, oh,
                        preferred_element_type=jnp.float32)
    bc = oh.sum(axis=1)
    bo = jnp.cumsum(bc, axis=0) - bc
    rank2 = (bo[:, None, :] + within).reshape(E, NW)
    rank = jnp.take_along_axis(rank2, bucket[:, None], axis=1)[:, 0]
    rank = rank.astype(jnp.int32)
    pos = jnp.where(rank < CAP, rank + bucket * CAP, NW * CAP)
    ccol = jnp.zeros((NW * CAP,), jnp.int32).at[pos].set(col, mode="drop")
    crow = jnp.full((NW * CAP,), SPILL, jnp.int32).at[pos].set(
        row - bucket * PR, mode="drop")
    counts = bc.sum(axis=0).astype(jnp.int32)
    cnt16 = jnp.repeat(jnp.minimum(counts, CAP), 16)
    return _gathermax(xn, ccol, crow, cnt16)


def _gelu_exact(x):
    return 0.5 * x * (1.0 + lax.erf(x * 0.7071067811865476))


def _ln(x, g, b, eps=1e-5):
    mu = jnp.mean(x, axis=-1, keepdims=True)
    var = jnp.mean((x - mu) ** 2, axis=-1, keepdims=True)
    return (x - mu) * lax.rsqrt(var + eps) * g + b


def _ln1_body(x_ref, g_ref, b_ref, o_ref):
    o_ref[...] = _ln(x_ref[...], g_ref[...], b_ref[...])


def _ln1(x2d, g1, be1):
    grid = (N // ROWS_BLK,)
    return pl.pallas_call(
        _ln1_body,
        grid=grid,
        in_specs=[
            pl.BlockSpec((ROWS_BLK, DIM), lambda i: (i, 0)),
            pl.BlockSpec((DIM,), lambda i: (0,)),
            pl.BlockSpec((DIM,), lambda i: (0,)),
        ],
        out_specs=pl.BlockSpec((ROWS_BLK, DIM), lambda i: (i, 0)),
        out_shape=jax.ShapeDtypeStruct((N, DIM), jnp.float32),
    )(x2d, g1, be1)


def _tail_body(aggr_ref, xn_ref, x_ref, W1_ref, bW1_ref, g2_ref, be2_ref,
               Wf1_ref, bf1_ref, Wf2_ref, bf2_ref, o_ref):
    aggr = aggr_ref[...]
    a = jnp.where(aggr == NEG, 0.0, aggr) - xn_ref[...]
    h = _gelu_exact(
        jnp.dot(a, W1_ref[...], preferred_element_type=jnp.float32) + bW1_ref[...])
    x1 = h + x_ref[...]
    xn2 = _ln(x1, g2_ref[...], be2_ref[...])
    hh = _gelu_exact(
        jnp.dot(xn2, Wf1_ref[...], preferred_element_type=jnp.float32) + bf1_ref[...])
    ff = jnp.dot(hh, Wf2_ref[...], preferred_element_type=jnp.float32) + bf2_ref[...]
    o_ref[...] = ff + x1


def _tail(aggr, xn, x2d, W1, bW1, g2, be2, Wf1, bf1, Wf2, bf2):
    grid = (N // ROWS_BLK,)
    row_spec = pl.BlockSpec((ROWS_BLK, DIM), lambda i: (i, 0))
    full = lambda shape: pl.BlockSpec(shape, lambda i: (0,) * len(shape))
    return pl.pallas_call(
        _tail_body,
        grid=grid,
        in_specs=[
            row_spec, row_spec, row_spec,
            full((DIM, DIM)), full((DIM,)), full((DIM,)), full((DIM,)),
            full((DIM, HID)), full((HID,)), full((HID, DIM)), full((DIM,)),
        ],
        out_specs=row_spec,
        out_shape=jax.ShapeDtypeStruct((N, DIM), jnp.float32),
    )(aggr, xn, x2d, W1, bW1, g2, be2, Wf1, bf1, Wf2, bf2)


def kernel(x, edge_index, g1, be1, W1, bW1, g2, be2, Wf1, bf1, Wf2, bf2):
    x2d = x.reshape(N, DIM)
    xn = _ln1(x2d, g1, be1)
    aggr = _aggregate(xn, edge_index[0], edge_index[1])[:N]
    out = _tail(aggr, xn, x2d, W1, bW1, g2, be2, Wf1, bf1, Wf2, bf2)
    return out.reshape(1, N, DIM)
